# baseline (speedup 1.0000x reference)
import jax
import jax.numpy as jnp
from jax.experimental import pallas as pl
from jax.experimental.pallas import tpu as pltpu

_B, _N, _C = 16, 5000, 200
_DUR_THRESH = 0.05


def _post_kernel(logits_ref, c_ref, lw_ref, vd_ref, off_ref,
                 scores_ref, labels_ref, t1_ref, t2_ref, mask_ref):
    b = pl.program_id(0)
    x = logits_ref[0]
    m = jnp.max(x, axis=-1)
    a = jnp.argmax(x, axis=-1).astype(jnp.int32)
    scores_ref[...] = jax.nn.sigmoid(m).reshape(1, 1, _N)
    labels_ref[...] = a.reshape(1, 1, _N)

    off = off_ref[b]
    vd = vd_ref[b]
    c = c_ref[...]
    half_w = 0.5 * jnp.exp(lw_ref[...])
    t1 = jnp.clip(c - half_w + off, 0.0, vd)
    t2 = jnp.clip(c + half_w + off, 0.0, vd)
    t1_ref[...] = t1
    t2_ref[...] = t2
    mask_ref[...] = (t2 - t1 > _DUR_THRESH).astype(jnp.int8)


_row_spec = pl.BlockSpec((1, 1, _N), lambda b: (b, 0, 0))


@jax.jit
def _run(pred_logits, c, lw, video_durations, offsets):
    out = pl.pallas_call(
        _post_kernel,
        grid=(_B,),
        in_specs=[
            pl.BlockSpec((1, _N, _C), lambda b: (b, 0, 0)),
            _row_spec,
            _row_spec,
            pl.BlockSpec(memory_space=pltpu.SMEM),
            pl.BlockSpec(memory_space=pltpu.SMEM),
        ],
        out_specs=[_row_spec] * 5,
        out_shape=[
            jax.ShapeDtypeStruct((_B, 1, _N), jnp.float32),
            jax.ShapeDtypeStruct((_B, 1, _N), jnp.int32),
            jax.ShapeDtypeStruct((_B, 1, _N), jnp.float32),
            jax.ShapeDtypeStruct((_B, 1, _N), jnp.float32),
            jax.ShapeDtypeStruct((_B, 1, _N), jnp.int8),
        ],
        compiler_params=pltpu.CompilerParams(
            dimension_semantics=("parallel",),
        ),
    )(pred_logits, c, lw, video_durations, offsets)
    return out


def kernel(pred_logits, pred_segments, video_durations, feature_durations, offsets):
    c = pred_segments[..., 0].reshape(_B, 1, _N)
    lw = pred_segments[..., 1].reshape(_B, 1, _N)
    scores, labels, t1, t2, mask = _run(
        pred_logits, c, lw, video_durations, offsets)
    segments = jnp.stack([t1.reshape(_B, _N), t2.reshape(_B, _N)], axis=-1)
    return (scores.reshape(_B, _N), labels.reshape(_B, _N), segments,
            mask.reshape(_B, _N).astype(jnp.bool_))
